# Initial kernel scaffold; baseline (speedup 1.0000x reference)
import jax
import jax.numpy as jnp
from jax.experimental import pallas as pl


def kernel(x, edge_index, batch, W1, W2, Wm1, bm1, Wm2, bm2):
    raise NotImplementedError("write your pallas kernel here")


# speedup vs baseline: 7.6604x; 7.6604x over previous
import functools
import jax
import jax.numpy as jnp
from jax import lax
from jax.experimental import pallas as pl
from jax.experimental.pallas import tpu as pltpu
from jax.experimental.pallas import tpu_sc as plsc

N = 10000
E = 320000
D = 128
G = 64

NC = 2
NS = 16
NW = NC * NS

NPAD = 10240
ROWS_PER_TILE = NPAD // NS
CH = 128
EPW = 10240
NCHUNK = EPW // CH
EPAD = NW * EPW

_mesh = plsc.VectorSubcoreMesh(
    core_axis_name="c", subcore_axis_name="s", num_cores=NC, num_subcores=NS)


@functools.partial(
    pl.kernel,
    out_type=jax.ShapeDtypeStruct((NC, NPAD), jnp.float32),
    mesh=_mesh,
    scratch_types=[
        pltpu.VMEM((NCHUNK, CH), jnp.int32),
        pltpu.VMEM((CH,), jnp.float32),
        pltpu.VMEM_SHARED((NPAD,), jnp.float32),
    ],
)
def _deg(dst_hbm, deg_out, dst_v, ones_v, deg_sh):
    cid = lax.axis_index("c")
    sid = lax.axis_index("s")
    wid = cid * NS + sid
    z16 = jnp.zeros((16,), jnp.float32)
    for i in range(CH // 16):
        ones_v[pl.ds(i * 16, 16)] = z16
    for i in range(ROWS_PER_TILE // CH):
        pltpu.sync_copy(ones_v, deg_sh.at[pl.ds(sid * ROWS_PER_TILE + i * CH, CH)])
    o16 = jnp.full((16,), 1.0, jnp.float32)
    for i in range(CH // 16):
        ones_v[pl.ds(i * 16, 16)] = o16
    pltpu.sync_copy(dst_hbm.at[wid], dst_v)
    plsc.subcore_barrier()

    def body(j, carry):
        pltpu.sync_copy(ones_v, deg_sh.at[dst_v.at[j]], add=True)
        return carry

    lax.fori_loop(0, NCHUNK, body, 0)
    plsc.subcore_barrier()
    pltpu.sync_copy(deg_sh.at[pl.ds(sid * ROWS_PER_TILE, ROWS_PER_TILE)],
                    deg_out.at[cid, pl.ds(sid * ROWS_PER_TILE, ROWS_PER_TILE)])


@functools.partial(
    pl.kernel,
    out_type=jax.ShapeDtypeStruct((NC, NPAD, D), jnp.float32),
    mesh=_mesh,
    scratch_types=[
        pltpu.VMEM((NCHUNK, CH), jnp.int32),
        pltpu.VMEM((NCHUNK, CH), jnp.int32),
        pltpu.VMEM((CH, D), jnp.float32),
        pltpu.VMEM_SHARED((NPAD, D), jnp.float32),
        pltpu.SemaphoreType.DMA,
    ],
)
def _scatter(hs_hbm, src_hbm, dst_hbm, out_hbm, src_v, dst_v, rows_v, acc_sh,
             sem):
    cid = lax.axis_index("c")
    sid = lax.axis_index("s")
    wid = cid * NS + sid
    z16 = jnp.zeros((16,), jnp.float32)

    def zero_rows(r, carry):
        for c in range(D // 16):
            rows_v[r, pl.ds(c * 16, 16)] = z16
        return carry

    lax.fori_loop(0, CH, zero_rows, 0)
    for i in range(ROWS_PER_TILE // CH):
        pltpu.sync_copy(rows_v,
                        acc_sh.at[pl.ds(sid * ROWS_PER_TILE + i * CH, CH)])
    pltpu.sync_copy(src_hbm.at[wid], src_v)
    pltpu.sync_copy(dst_hbm.at[wid], dst_v)
    plsc.subcore_barrier()

    def body(j, carry):
        pltpu.async_copy(hs_hbm.at[src_v.at[j]], rows_v, sem).wait()
        pltpu.sync_copy(rows_v, acc_sh.at[dst_v.at[j]], add=True)
        return carry

    lax.fori_loop(0, NCHUNK, body, 0)
    plsc.subcore_barrier()
    pltpu.sync_copy(acc_sh.at[pl.ds(sid * ROWS_PER_TILE, ROWS_PER_TILE)],
                    out_hbm.at[cid, pl.ds(sid * ROWS_PER_TILE, ROWS_PER_TILE)])


BM = 256
NB = NPAD // BM


def _mm1_body(d0_ref, d1_ref, x_ref, w_ref, o_ref):
    dinv = lax.rsqrt(d0_ref[...] + d1_ref[...] + 1.0)
    o_ref[...] = jnp.dot(x_ref[...], w_ref[...],
                         preferred_element_type=jnp.float32) * dinv


def _mm1(d0, d1, xp, w1):
    return pl.pallas_call(
        _mm1_body,
        grid=(NB,),
        in_specs=[
            pl.BlockSpec((BM, 1), lambda i: (i, 0)),
            pl.BlockSpec((BM, 1), lambda i: (i, 0)),
            pl.BlockSpec((BM, D), lambda i: (i, 0)),
            pl.BlockSpec((D, D), lambda i: (0, 0)),
        ],
        out_specs=pl.BlockSpec((BM, D), lambda i: (i, 0)),
        out_shape=jax.ShapeDtypeStruct((NPAD, D), jnp.float32),
    )(d0, d1, xp, w1)


def _mm2_body(d0_ref, d1_ref, p0_ref, p1_ref, hs_ref, w_ref, o_ref):
    dinv = lax.rsqrt(d0_ref[...] + d1_ref[...] + 1.0)
    h = jnp.maximum((p0_ref[...] + p1_ref[...] + hs_ref[...]) * dinv, 0.0)
    o_ref[...] = jnp.dot(h, w_ref[...],
                         preferred_element_type=jnp.float32) * dinv


def _mm2(d0, d1, p0, p1, hs1, w2):
    return pl.pallas_call(
        _mm2_body,
        grid=(NB,),
        in_specs=[
            pl.BlockSpec((BM, 1), lambda i: (i, 0)),
            pl.BlockSpec((BM, 1), lambda i: (i, 0)),
            pl.BlockSpec((BM, D), lambda i: (i, 0)),
            pl.BlockSpec((BM, D), lambda i: (i, 0)),
            pl.BlockSpec((BM, D), lambda i: (i, 0)),
            pl.BlockSpec((D, D), lambda i: (0, 0)),
        ],
        out_specs=pl.BlockSpec((BM, D), lambda i: (i, 0)),
        out_shape=jax.ShapeDtypeStruct((NPAD, D), jnp.float32),
    )(d0, d1, p0, p1, hs1, w2)


def _pool_body(d0_ref, d1_ref, p0_ref, p1_ref, hs_ref, b_ref, wm1_ref,
               bm1_ref, wm2_ref, bm2_ref, o_ref, pooled_acc, cnt_acc):
    i = pl.program_id(0)

    @pl.when(i == 0)
    def _():
        pooled_acc[...] = jnp.zeros((G, D), jnp.float32)
        cnt_acc[...] = jnp.zeros((G, D), jnp.float32)

    dinv = lax.rsqrt(d0_ref[...] + d1_ref[...] + 1.0)
    h2 = (p0_ref[...] + p1_ref[...] + hs_ref[...]) * dinv
    ids = jnp.broadcast_to(b_ref[...], (G, BM))
    oht = (lax.broadcasted_iota(jnp.int32, (G, BM), 0) == ids)
    oht = oht.astype(jnp.float32)
    pooled_acc[...] += jnp.dot(oht, h2, preferred_element_type=jnp.float32)
    cnt_acc[...] += jnp.dot(oht, jnp.ones((BM, D), jnp.float32),
                            preferred_element_type=jnp.float32)

    @pl.when(i == NB - 1)
    def _():
        mean = pooled_acc[...] / jnp.maximum(cnt_acc[...], 1.0)
        z = jnp.maximum(
            jnp.dot(mean, wm1_ref[...], preferred_element_type=jnp.float32)
            + bm1_ref[...], 0.0)
        o_ref[...] = jnp.dot(z, wm2_ref[...],
                             preferred_element_type=jnp.float32) + bm2_ref[...]


def _pool(d0, d1, p0, p1, hs2, batchr, wm1, bm1, wm2, bm2):
    return pl.pallas_call(
        _pool_body,
        grid=(NB,),
        in_specs=[
            pl.BlockSpec((BM, 1), lambda i: (i, 0)),
            pl.BlockSpec((BM, 1), lambda i: (i, 0)),
            pl.BlockSpec((BM, D), lambda i: (i, 0)),
            pl.BlockSpec((BM, D), lambda i: (i, 0)),
            pl.BlockSpec((BM, D), lambda i: (i, 0)),
            pl.BlockSpec((1, BM), lambda i: (0, i)),
            pl.BlockSpec((D, D), lambda i: (0, 0)),
            pl.BlockSpec((1, D), lambda i: (0, 0)),
            pl.BlockSpec((D, D), lambda i: (0, 0)),
            pl.BlockSpec((1, D), lambda i: (0, 0)),
        ],
        out_specs=pl.BlockSpec((G, D), lambda i: (0, 0)),
        out_shape=jax.ShapeDtypeStruct((G, D), jnp.float32),
        scratch_shapes=[
            pltpu.VMEM((G, D), jnp.float32),
            pltpu.VMEM((G, D), jnp.float32),
        ],
    )(d0, d1, p0, p1, hs2, batchr, wm1, bm1, wm2, bm2)


def kernel(x, edge_index, batch, W1, W2, Wm1, bm1, Wm2, bm2):
    xp = jnp.pad(x, ((0, NPAD - N), (0, 0)))
    src = jnp.pad(edge_index[0].astype(jnp.int32), (0, EPAD - E),
                  constant_values=N).reshape(NW, NCHUNK, CH)
    dst = jnp.pad(edge_index[1].astype(jnp.int32), (0, EPAD - E),
                  constant_values=N).reshape(NW, NCHUNK, CH)
    batchr = jnp.pad(batch.astype(jnp.int32), (0, NPAD - N),
                     constant_values=G).reshape(1, NPAD)

    deg = _deg(dst)
    d0 = deg[0].reshape(NPAD, 1)
    d1 = deg[1].reshape(NPAD, 1)

    hs1 = _mm1(d0, d1, xp, W1)
    p = _scatter(hs1, src, dst)
    hs2 = _mm2(d0, d1, p[0], p[1], hs1, W2)
    q = _scatter(hs2, src, dst)
    return _pool(d0, d1, q[0], q[1], hs2, batchr,
                 Wm1, bm1.reshape(1, D), Wm2, bm2.reshape(1, D))


# speedup vs baseline: 8.4057x; 1.0973x over previous
import functools
import jax
import jax.numpy as jnp
from jax import lax
from jax.experimental import pallas as pl
from jax.experimental.pallas import tpu as pltpu
from jax.experimental.pallas import tpu_sc as plsc

N = 10000
E = 320000
D = 128
G = 64

NC = 2
NS = 16
NW = NC * NS

NPAD = 10240
ROWS_PER_TILE = NPAD // NS
CH = 128
EPW = 10240
NCHUNK = EPW // CH
EPAD = NW * EPW

_mesh = plsc.VectorSubcoreMesh(
    core_axis_name="c", subcore_axis_name="s", num_cores=NC, num_subcores=NS)


@functools.partial(
    pl.kernel,
    out_type=jax.ShapeDtypeStruct((NC, NPAD), jnp.float32),
    mesh=_mesh,
    scratch_types=[
        pltpu.VMEM((NCHUNK, CH), jnp.int32),
        pltpu.VMEM((CH,), jnp.float32),
        pltpu.VMEM_SHARED((NPAD,), jnp.float32),
    ],
)
def _deg(dst_hbm, deg_out, dst_v, ones_v, deg_sh):
    cid = lax.axis_index("c")
    sid = lax.axis_index("s")
    wid = cid * NS + sid
    z16 = jnp.zeros((16,), jnp.float32)
    for i in range(CH // 16):
        ones_v[pl.ds(i * 16, 16)] = z16
    for i in range(ROWS_PER_TILE // CH):
        pltpu.sync_copy(ones_v, deg_sh.at[pl.ds(sid * ROWS_PER_TILE + i * CH, CH)])
    o16 = jnp.full((16,), 1.0, jnp.float32)
    for i in range(CH // 16):
        ones_v[pl.ds(i * 16, 16)] = o16
    pltpu.sync_copy(dst_hbm.at[wid], dst_v)
    plsc.subcore_barrier()

    def body(j, carry):
        pltpu.sync_copy(ones_v, deg_sh.at[dst_v.at[j]], add=True)
        return carry

    lax.fori_loop(0, NCHUNK, body, 0)
    plsc.subcore_barrier()
    pltpu.sync_copy(deg_sh.at[pl.ds(sid * ROWS_PER_TILE, ROWS_PER_TILE)],
                    deg_out.at[cid, pl.ds(sid * ROWS_PER_TILE, ROWS_PER_TILE)])


HALF = NCHUNK // 2


@functools.partial(
    pl.kernel,
    out_type=jax.ShapeDtypeStruct((NC, NPAD, D), jnp.float32),
    mesh=_mesh,
    scratch_types=[
        pltpu.VMEM((NCHUNK, CH), jnp.int32),
        pltpu.VMEM((HALF, CH), jnp.int32),
        pltpu.VMEM((2, CH, D), jnp.float32),
        pltpu.VMEM_SHARED((NPAD, D), jnp.float32),
        pltpu.SemaphoreType.DMA,
        pltpu.SemaphoreType.DMA,
    ],
)
def _scatter(hs_hbm, src_hbm, dst_hbm, out_hbm, src_v, dst_hv, rows, acc_sh,
             gsem0, gsem1):
    cid = lax.axis_index("c")
    sid = lax.axis_index("s")
    wid = cid * NS + sid
    z16 = jnp.zeros((16,), jnp.float32)

    def zero_rows(r, carry):
        for c in range(D // 16):
            rows[0, r, pl.ds(c * 16, 16)] = z16
        return carry

    lax.fori_loop(0, CH, zero_rows, 0)

    def zinit(i, carry):
        pltpu.sync_copy(rows.at[0],
                        acc_sh.at[pl.ds(sid * ROWS_PER_TILE + i * CH, CH)])
        return carry

    lax.fori_loop(0, ROWS_PER_TILE // CH, zinit, 0)
    pltpu.sync_copy(src_hbm.at[wid], src_v)
    pltpu.sync_copy(dst_hbm.at[wid, pl.ds(0, HALF)], dst_hv)
    pltpu.async_copy(hs_hbm.at[src_v.at[0]], rows.at[0], gsem0)
    pltpu.async_copy(hs_hbm.at[src_v.at[1]], rows.at[1], gsem1)
    plsc.subcore_barrier()

    def body(jj, carry):
        j0 = 2 * jj
        j1 = j0 + 1

        @pl.when(jj == HALF // 2)
        def _():
            pltpu.sync_copy(dst_hbm.at[wid, pl.ds(HALF, HALF)], dst_hv)

        pltpu.make_async_copy(hs_hbm.at[src_v.at[j0]], rows.at[0],
                              gsem0).wait()
        pltpu.sync_copy(rows.at[0], acc_sh.at[dst_hv.at[lax.rem(j0, HALF)]],
                        add=True)
        pltpu.async_copy(hs_hbm.at[src_v.at[lax.rem(j0 + 2, NCHUNK)]],
                         rows.at[0], gsem0)
        pltpu.make_async_copy(hs_hbm.at[src_v.at[j1]], rows.at[1],
                              gsem1).wait()
        pltpu.sync_copy(rows.at[1], acc_sh.at[dst_hv.at[lax.rem(j1, HALF)]],
                        add=True)
        pltpu.async_copy(hs_hbm.at[src_v.at[lax.rem(j1 + 2, NCHUNK)]],
                         rows.at[1], gsem1)
        return carry

    lax.fori_loop(0, NCHUNK // 2, body, 0)
    pltpu.make_async_copy(hs_hbm.at[src_v.at[0]], rows.at[0], gsem0).wait()
    pltpu.make_async_copy(hs_hbm.at[src_v.at[1]], rows.at[1], gsem1).wait()
    plsc.subcore_barrier()
    pltpu.sync_copy(acc_sh.at[pl.ds(sid * ROWS_PER_TILE, ROWS_PER_TILE)],
                    out_hbm.at[cid, pl.ds(sid * ROWS_PER_TILE, ROWS_PER_TILE)])


BM = 256
NB = NPAD // BM


def _mm1_body(d0_ref, d1_ref, x_ref, w_ref, o_ref):
    dinv = lax.rsqrt(d0_ref[...] + d1_ref[...] + 1.0)
    o_ref[...] = jnp.dot(x_ref[...], w_ref[...],
                         preferred_element_type=jnp.float32) * dinv


def _mm1(d0, d1, xp, w1):
    return pl.pallas_call(
        _mm1_body,
        grid=(NB,),
        in_specs=[
            pl.BlockSpec((BM, 1), lambda i: (i, 0)),
            pl.BlockSpec((BM, 1), lambda i: (i, 0)),
            pl.BlockSpec((BM, D), lambda i: (i, 0)),
            pl.BlockSpec((D, D), lambda i: (0, 0)),
        ],
        out_specs=pl.BlockSpec((BM, D), lambda i: (i, 0)),
        out_shape=jax.ShapeDtypeStruct((NPAD, D), jnp.float32),
    )(d0, d1, xp, w1)


def _mm2_body(d0_ref, d1_ref, p0_ref, p1_ref, hs_ref, w_ref, o_ref):
    dinv = lax.rsqrt(d0_ref[...] + d1_ref[...] + 1.0)
    h = jnp.maximum((p0_ref[...] + p1_ref[...] + hs_ref[...]) * dinv, 0.0)
    o_ref[...] = jnp.dot(h, w_ref[...],
                         preferred_element_type=jnp.float32) * dinv


def _mm2(d0, d1, p0, p1, hs1, w2):
    return pl.pallas_call(
        _mm2_body,
        grid=(NB,),
        in_specs=[
            pl.BlockSpec((BM, 1), lambda i: (i, 0)),
            pl.BlockSpec((BM, 1), lambda i: (i, 0)),
            pl.BlockSpec((BM, D), lambda i: (i, 0)),
            pl.BlockSpec((BM, D), lambda i: (i, 0)),
            pl.BlockSpec((BM, D), lambda i: (i, 0)),
            pl.BlockSpec((D, D), lambda i: (0, 0)),
        ],
        out_specs=pl.BlockSpec((BM, D), lambda i: (i, 0)),
        out_shape=jax.ShapeDtypeStruct((NPAD, D), jnp.float32),
    )(d0, d1, p0, p1, hs1, w2)


def _pool_body(d0_ref, d1_ref, p0_ref, p1_ref, hs_ref, b_ref, wm1_ref,
               bm1_ref, wm2_ref, bm2_ref, o_ref, pooled_acc, cnt_acc):
    i = pl.program_id(0)

    @pl.when(i == 0)
    def _():
        pooled_acc[...] = jnp.zeros((G, D), jnp.float32)
        cnt_acc[...] = jnp.zeros((G, D), jnp.float32)

    dinv = lax.rsqrt(d0_ref[...] + d1_ref[...] + 1.0)
    h2 = (p0_ref[...] + p1_ref[...] + hs_ref[...]) * dinv
    ids = jnp.broadcast_to(b_ref[...], (G, BM))
    oht = (lax.broadcasted_iota(jnp.int32, (G, BM), 0) == ids)
    oht = oht.astype(jnp.float32)
    pooled_acc[...] += jnp.dot(oht, h2, preferred_element_type=jnp.float32)
    cnt_acc[...] += jnp.dot(oht, jnp.ones((BM, D), jnp.float32),
                            preferred_element_type=jnp.float32)

    @pl.when(i == NB - 1)
    def _():
        mean = pooled_acc[...] / jnp.maximum(cnt_acc[...], 1.0)
        z = jnp.maximum(
            jnp.dot(mean, wm1_ref[...], preferred_element_type=jnp.float32)
            + bm1_ref[...], 0.0)
        o_ref[...] = jnp.dot(z, wm2_ref[...],
                             preferred_element_type=jnp.float32) + bm2_ref[...]


def _pool(d0, d1, p0, p1, hs2, batchr, wm1, bm1, wm2, bm2):
    return pl.pallas_call(
        _pool_body,
        grid=(NB,),
        in_specs=[
            pl.BlockSpec((BM, 1), lambda i: (i, 0)),
            pl.BlockSpec((BM, 1), lambda i: (i, 0)),
            pl.BlockSpec((BM, D), lambda i: (i, 0)),
            pl.BlockSpec((BM, D), lambda i: (i, 0)),
            pl.BlockSpec((BM, D), lambda i: (i, 0)),
            pl.BlockSpec((1, BM), lambda i: (0, i)),
            pl.BlockSpec((D, D), lambda i: (0, 0)),
            pl.BlockSpec((1, D), lambda i: (0, 0)),
            pl.BlockSpec((D, D), lambda i: (0, 0)),
            pl.BlockSpec((1, D), lambda i: (0, 0)),
        ],
        out_specs=pl.BlockSpec((G, D), lambda i: (0, 0)),
        out_shape=jax.ShapeDtypeStruct((G, D), jnp.float32),
        scratch_shapes=[
            pltpu.VMEM((G, D), jnp.float32),
            pltpu.VMEM((G, D), jnp.float32),
        ],
    )(d0, d1, p0, p1, hs2, batchr, wm1, bm1, wm2, bm2)


def kernel(x, edge_index, batch, W1, W2, Wm1, bm1, Wm2, bm2):
    xp = jnp.pad(x, ((0, NPAD - N), (0, 0)))
    src = jnp.pad(edge_index[0].astype(jnp.int32), (0, EPAD - E),
                  constant_values=N).reshape(NW, NCHUNK, CH)
    dst = jnp.pad(edge_index[1].astype(jnp.int32), (0, EPAD - E),
                  constant_values=N).reshape(NW, NCHUNK, CH)
    batchr = jnp.pad(batch.astype(jnp.int32), (0, NPAD - N),
                     constant_values=G).reshape(1, NPAD)

    deg = _deg(dst)
    d0 = deg[0].reshape(NPAD, 1)
    d1 = deg[1].reshape(NPAD, 1)

    hs1 = _mm1(d0, d1, xp, W1)
    p = _scatter(hs1, src, dst)
    hs2 = _mm2(d0, d1, p[0], p[1], hs1, W2)
    q = _scatter(hs2, src, dst)
    return _pool(d0, d1, q[0], q[1], hs2, batchr,
                 Wm1, bm1.reshape(1, D), Wm2, bm2.reshape(1, D))


# speedup vs baseline: 11.3283x; 1.3477x over previous
import functools
import jax
import jax.numpy as jnp
from jax import lax
from jax.experimental import pallas as pl
from jax.experimental.pallas import tpu as pltpu
from jax.experimental.pallas import tpu_sc as plsc

N = 10000
E = 320000
D = 128
G = 64

NC = 2
NS = 16
NW = NC * NS

NPAD = 10240
ROWS_PER_TILE = NPAD // NS
CH = 128
APAIR = 158
A0 = 28
A1 = APAIR - A0
GB = 16
APAD = ((A1 + GB - 1) // GB) * GB

_mesh = plsc.VectorSubcoreMesh(
    core_axis_name="c", subcore_axis_name="s", num_cores=NC, num_subcores=NS)


@functools.partial(
    pl.kernel,
    out_type=jax.ShapeDtypeStruct((NC, NPAD), jnp.float32),
    mesh=_mesh,
    scratch_types=[
        pltpu.VMEM((APAD, CH), jnp.int32),
        pltpu.VMEM((CH,), jnp.float32),
        pltpu.VMEM_SHARED((NPAD,), jnp.float32),
    ],
)
def _deg(dst_hbm, deg_out, dst_v, ones_v, deg_sh):
    cid = lax.axis_index("c")
    sid = lax.axis_index("s")
    wid = cid * NS + sid
    nch = jnp.where(cid == 0, A0, A1)
    z16 = jnp.zeros((16,), jnp.float32)
    for i in range(CH // 16):
        ones_v[pl.ds(i * 16, 16)] = z16
    for i in range(ROWS_PER_TILE // CH):
        pltpu.sync_copy(ones_v, deg_sh.at[pl.ds(sid * ROWS_PER_TILE + i * CH, CH)])
    o16 = jnp.full((16,), 1.0, jnp.float32)
    for i in range(CH // 16):
        ones_v[pl.ds(i * 16, 16)] = o16
    pltpu.sync_copy(dst_hbm.at[wid], dst_v)
    plsc.subcore_barrier()

    def body(j, carry):
        pltpu.sync_copy(ones_v, deg_sh.at[dst_v.at[j]], add=True)
        return carry

    lax.fori_loop(0, nch, body, 0)
    plsc.subcore_barrier()
    pltpu.sync_copy(deg_sh.at[pl.ds(sid * ROWS_PER_TILE, ROWS_PER_TILE)],
                    deg_out.at[cid, pl.ds(sid * ROWS_PER_TILE, ROWS_PER_TILE)])


@functools.partial(
    pl.kernel,
    out_type=jax.ShapeDtypeStruct((NC, NPAD, D), jnp.float32),
    mesh=_mesh,
    scratch_types=[
        pltpu.VMEM((GB, CH), jnp.int32),
        pltpu.VMEM((GB, CH), jnp.int32),
        pltpu.VMEM((2, CH, D), jnp.float32),
        pltpu.VMEM_SHARED((NPAD, D), jnp.float32),
        pltpu.SemaphoreType.DMA,
        pltpu.SemaphoreType.DMA,
    ],
)
def _scatter(hs_hbm, src_hbm, dst_hbm, out_hbm, srcb, dstb, rows, acc_sh,
             gsem0, gsem1):
    cid = lax.axis_index("c")
    sid = lax.axis_index("s")
    wid = cid * NS + sid
    nch = jnp.where(cid == 0, A0, A1)
    ngroups = (nch + GB - 1) // GB
    z16 = jnp.zeros((16,), jnp.float32)

    def zero_rows(r, carry):
        for c in range(D // 16):
            rows[0, r, pl.ds(c * 16, 16)] = z16
        return carry

    lax.fori_loop(0, CH, zero_rows, 0)

    def zinit(i, carry):
        pltpu.sync_copy(rows.at[0],
                        acc_sh.at[pl.ds(sid * ROWS_PER_TILE + i * CH, CH)])
        return carry

    lax.fori_loop(0, ROWS_PER_TILE // CH, zinit, 0)
    plsc.subcore_barrier()

    def group(g, carry):
        gc = jnp.minimum(nch - g * GB, GB)
        pltpu.sync_copy(src_hbm.at[wid, pl.ds(g * GB, GB)], srcb)
        pltpu.sync_copy(dst_hbm.at[wid, pl.ds(g * GB, GB)], dstb)
        pltpu.async_copy(hs_hbm.at[srcb.at[0]], rows.at[0], gsem0)
        pltpu.async_copy(hs_hbm.at[srcb.at[1]], rows.at[1], gsem1)

        def body(jj, carry2):
            j0 = 2 * jj
            j1 = j0 + 1
            pltpu.make_async_copy(hs_hbm.at[srcb.at[j0]], rows.at[0],
                                  gsem0).wait()
            pltpu.sync_copy(rows.at[0], acc_sh.at[dstb.at[j0]], add=True)
            pltpu.async_copy(hs_hbm.at[srcb.at[lax.rem(j0 + 2, gc)]],
                             rows.at[0], gsem0)
            pltpu.make_async_copy(hs_hbm.at[srcb.at[j1]], rows.at[1],
                                  gsem1).wait()
            pltpu.sync_copy(rows.at[1], acc_sh.at[dstb.at[j1]], add=True)
            pltpu.async_copy(hs_hbm.at[srcb.at[lax.rem(j1 + 2, gc)]],
                             rows.at[1], gsem1)
            return carry2

        lax.fori_loop(0, gc // 2, body, 0)
        pltpu.make_async_copy(hs_hbm.at[srcb.at[0]], rows.at[0], gsem0).wait()
        pltpu.make_async_copy(hs_hbm.at[srcb.at[1]], rows.at[1], gsem1).wait()
        return carry

    lax.fori_loop(0, ngroups, group, 0)
    plsc.subcore_barrier()
    pltpu.sync_copy(acc_sh.at[pl.ds(sid * ROWS_PER_TILE, ROWS_PER_TILE)],
                    out_hbm.at[cid, pl.ds(sid * ROWS_PER_TILE, ROWS_PER_TILE)])


BM = 256
NB = NPAD // BM


def _mm1_body(d0_ref, d1_ref, x_ref, w_ref, o_ref):
    dinv = lax.rsqrt(d0_ref[...] + d1_ref[...] + 1.0)
    o_ref[...] = jnp.dot(x_ref[...], w_ref[...],
                         preferred_element_type=jnp.float32) * dinv


def _mm1(d0, d1, xp, w1):
    return pl.pallas_call(
        _mm1_body,
        grid=(NB,),
        in_specs=[
            pl.BlockSpec((BM, 1), lambda i: (i, 0)),
            pl.BlockSpec((BM, 1), lambda i: (i, 0)),
            pl.BlockSpec((BM, D), lambda i: (i, 0)),
            pl.BlockSpec((D, D), lambda i: (0, 0)),
        ],
        out_specs=pl.BlockSpec((BM, D), lambda i: (i, 0)),
        out_shape=jax.ShapeDtypeStruct((NPAD, D), jnp.float32),
    )(d0, d1, xp, w1)


def _mm2_body(d0_ref, d1_ref, p0_ref, p1_ref, hs_ref, w_ref, o_ref):
    dinv = lax.rsqrt(d0_ref[...] + d1_ref[...] + 1.0)
    h = jnp.maximum((p0_ref[...] + p1_ref[...] + hs_ref[...]) * dinv, 0.0)
    o_ref[...] = jnp.dot(h, w_ref[...],
                         preferred_element_type=jnp.float32) * dinv


def _mm2(d0, d1, p0, p1, hs1, w2):
    return pl.pallas_call(
        _mm2_body,
        grid=(NB,),
        in_specs=[
            pl.BlockSpec((BM, 1), lambda i: (i, 0)),
            pl.BlockSpec((BM, 1), lambda i: (i, 0)),
            pl.BlockSpec((BM, D), lambda i: (i, 0)),
            pl.BlockSpec((BM, D), lambda i: (i, 0)),
            pl.BlockSpec((BM, D), lambda i: (i, 0)),
            pl.BlockSpec((D, D), lambda i: (0, 0)),
        ],
        out_specs=pl.BlockSpec((BM, D), lambda i: (i, 0)),
        out_shape=jax.ShapeDtypeStruct((NPAD, D), jnp.float32),
    )(d0, d1, p0, p1, hs1, w2)


def _pool_body(d0_ref, d1_ref, p0_ref, p1_ref, hs_ref, b_ref, wm1_ref,
               bm1_ref, wm2_ref, bm2_ref, o_ref, pooled_acc, cnt_acc):
    i = pl.program_id(0)

    @pl.when(i == 0)
    def _():
        pooled_acc[...] = jnp.zeros((G, D), jnp.float32)
        cnt_acc[...] = jnp.zeros((G, D), jnp.float32)

    dinv = lax.rsqrt(d0_ref[...] + d1_ref[...] + 1.0)
    h2 = (p0_ref[...] + p1_ref[...] + hs_ref[...]) * dinv
    ids = jnp.broadcast_to(b_ref[...], (G, BM))
    oht = (lax.broadcasted_iota(jnp.int32, (G, BM), 0) == ids)
    oht = oht.astype(jnp.float32)
    pooled_acc[...] += jnp.dot(oht, h2, preferred_element_type=jnp.float32)
    cnt_acc[...] += jnp.dot(oht, jnp.ones((BM, D), jnp.float32),
                            preferred_element_type=jnp.float32)

    @pl.when(i == NB - 1)
    def _():
        mean = pooled_acc[...] / jnp.maximum(cnt_acc[...], 1.0)
        z = jnp.maximum(
            jnp.dot(mean, wm1_ref[...], preferred_element_type=jnp.float32)
            + bm1_ref[...], 0.0)
        o_ref[...] = jnp.dot(z, wm2_ref[...],
                             preferred_element_type=jnp.float32) + bm2_ref[...]


def _pool(d0, d1, p0, p1, hs2, batchr, wm1, bm1, wm2, bm2):
    return pl.pallas_call(
        _pool_body,
        grid=(NB,),
        in_specs=[
            pl.BlockSpec((BM, 1), lambda i: (i, 0)),
            pl.BlockSpec((BM, 1), lambda i: (i, 0)),
            pl.BlockSpec((BM, D), lambda i: (i, 0)),
            pl.BlockSpec((BM, D), lambda i: (i, 0)),
            pl.BlockSpec((BM, D), lambda i: (i, 0)),
            pl.BlockSpec((1, BM), lambda i: (0, i)),
            pl.BlockSpec((D, D), lambda i: (0, 0)),
            pl.BlockSpec((1, D), lambda i: (0, 0)),
            pl.BlockSpec((D, D), lambda i: (0, 0)),
            pl.BlockSpec((1, D), lambda i: (0, 0)),
        ],
        out_specs=pl.BlockSpec((G, D), lambda i: (0, 0)),
        out_shape=jax.ShapeDtypeStruct((G, D), jnp.float32),
        scratch_shapes=[
            pltpu.VMEM((G, D), jnp.float32),
            pltpu.VMEM((G, D), jnp.float32),
        ],
    )(d0, d1, p0, p1, hs2, batchr, wm1, bm1, wm2, bm2)


def kernel(x, edge_index, batch, W1, W2, Wm1, bm1, Wm2, bm2):
    xp = jnp.pad(x, ((0, NPAD - N), (0, 0)))
    def _split(idx):
        a = jnp.pad(idx.astype(jnp.int32), (0, NS * APAIR * CH - E),
                    constant_values=N).reshape(NS, APAIR, CH)
        a0 = jnp.pad(a[:, :A0], ((0, 0), (0, APAD - A0), (0, 0)),
                     constant_values=N)
        a1 = jnp.pad(a[:, A0:], ((0, 0), (0, APAD - A1), (0, 0)),
                     constant_values=N)
        return jnp.concatenate([a0, a1], axis=0)

    src = _split(edge_index[0])
    dst = _split(edge_index[1])
    batchr = jnp.pad(batch.astype(jnp.int32), (0, NPAD - N),
                     constant_values=G).reshape(1, NPAD)

    deg = _deg(dst)
    d0 = deg[0].reshape(NPAD, 1)
    d1 = deg[1].reshape(NPAD, 1)

    hs1 = _mm1(d0, d1, xp, W1)
    p = _scatter(hs1, src, dst)
    hs2 = _mm2(d0, d1, p[0], p[1], hs1, W2)
    q = _scatter(hs2, src, dst)
    return _pool(d0, d1, q[0], q[1], hs2, batchr,
                 Wm1, bm1.reshape(1, D), Wm2, bm2.reshape(1, D))


# speedup vs baseline: 12.2333x; 1.0799x over previous
import functools
import jax
import jax.numpy as jnp
from jax import lax
from jax.experimental import pallas as pl
from jax.experimental.pallas import tpu as pltpu
from jax.experimental.pallas import tpu_sc as plsc

N = 10000
E = 320000
D = 128
G = 64

NC = 2
NS = 16
NW = NC * NS

NPAD = 10240
ROWS_PER_TILE = NPAD // NS
CH = 128
APT = 158
GB = 16
APAD = ((APT + GB - 1) // GB) * GB

_mesh = plsc.VectorSubcoreMesh(
    core_axis_name="c", subcore_axis_name="s", num_cores=NC, num_subcores=NS)


@functools.partial(
    pl.kernel,
    out_type=jax.ShapeDtypeStruct((NPAD,), jnp.float32),
    mesh=_mesh,
    scratch_types=[
        pltpu.VMEM((APAD, CH), jnp.int32),
        pltpu.VMEM((CH,), jnp.float32),
        pltpu.VMEM_SHARED((NPAD,), jnp.float32),
    ],
)
def _deg(dst_hbm, deg_out, dst_v, ones_v, deg_sh):
    cid = lax.axis_index("c")
    sid = lax.axis_index("s")

    @pl.when(cid == 1)
    def _():
        z16 = jnp.zeros((16,), jnp.float32)
        for i in range(CH // 16):
            ones_v[pl.ds(i * 16, 16)] = z16
        for i in range(ROWS_PER_TILE // CH):
            pltpu.sync_copy(
                ones_v, deg_sh.at[pl.ds(sid * ROWS_PER_TILE + i * CH, CH)])
        o16 = jnp.full((16,), 1.0, jnp.float32)
        for i in range(CH // 16):
            ones_v[pl.ds(i * 16, 16)] = o16
        pltpu.sync_copy(dst_hbm.at[sid], dst_v)
        plsc.subcore_barrier()

        def body(j, carry):
            pltpu.sync_copy(ones_v, deg_sh.at[dst_v.at[j]], add=True)
            return carry

        lax.fori_loop(0, APT, body, 0)
        plsc.subcore_barrier()
        pltpu.sync_copy(deg_sh.at[pl.ds(sid * ROWS_PER_TILE, ROWS_PER_TILE)],
                        deg_out.at[pl.ds(sid * ROWS_PER_TILE, ROWS_PER_TILE)])


@functools.partial(
    pl.kernel,
    out_type=jax.ShapeDtypeStruct((NPAD, D), jnp.float32),
    mesh=_mesh,
    scratch_types=[
        pltpu.VMEM((GB, CH), jnp.int32),
        pltpu.VMEM((GB, CH), jnp.int32),
        pltpu.VMEM((2, CH, D), jnp.float32),
        pltpu.VMEM_SHARED((NPAD, D), jnp.float32),
        pltpu.SemaphoreType.DMA,
        pltpu.SemaphoreType.DMA,
        pltpu.SemaphoreType.DMA,
        pltpu.SemaphoreType.DMA,
    ],
)
def _scatter(hs_hbm, src_hbm, dst_hbm, out_hbm, srcb, dstb, rows, acc_sh,
             gsem0, gsem1, gsem2, gsem3):
    cid = lax.axis_index("c")
    sid = lax.axis_index("s")

    @pl.when(cid == 1)
    def _():
        z16 = jnp.zeros((16,), jnp.float32)

        def zero_rows(r, carry):
            for c in range(D // 16):
                rows[0, r, pl.ds(c * 16, 16)] = z16
            return carry

        lax.fori_loop(0, CH, zero_rows, 0)

        def zinit(i, carry):
            pltpu.sync_copy(rows.at[0],
                            acc_sh.at[pl.ds(sid * ROWS_PER_TILE + i * CH, CH)])
            return carry

        lax.fori_loop(0, ROWS_PER_TILE // CH, zinit, 0)
        plsc.subcore_barrier()

        def group(g, carry):
            gc = jnp.minimum(APT - g * GB, GB)
            pltpu.sync_copy(src_hbm.at[sid, pl.ds(g * GB, GB)], srcb)
            pltpu.sync_copy(dst_hbm.at[sid, pl.ds(g * GB, GB)], dstb)
            H2 = CH // 2

            def fire(j, slot, semA, semB):
                pltpu.async_copy(hs_hbm.at[srcb.at[j, pl.ds(0, H2)]],
                                 rows.at[slot, pl.ds(0, H2)], semA)
                pltpu.async_copy(hs_hbm.at[srcb.at[j, pl.ds(H2, H2)]],
                                 rows.at[slot, pl.ds(H2, H2)], semB)

            def wait(j, slot, semA, semB):
                pltpu.make_async_copy(hs_hbm.at[srcb.at[j, pl.ds(0, H2)]],
                                      rows.at[slot, pl.ds(0, H2)], semA).wait()
                pltpu.make_async_copy(hs_hbm.at[srcb.at[j, pl.ds(H2, H2)]],
                                      rows.at[slot, pl.ds(H2, H2)], semB).wait()

            fire(0, 0, gsem0, gsem1)
            fire(1, 1, gsem2, gsem3)

            def body(jj, carry2):
                j0 = 2 * jj
                j1 = j0 + 1
                wait(j0, 0, gsem0, gsem1)
                pltpu.sync_copy(rows.at[0], acc_sh.at[dstb.at[j0]], add=True)
                fire(lax.rem(j0 + 2, gc), 0, gsem0, gsem1)
                wait(j1, 1, gsem2, gsem3)
                pltpu.sync_copy(rows.at[1], acc_sh.at[dstb.at[j1]], add=True)
                fire(lax.rem(j1 + 2, gc), 1, gsem2, gsem3)
                return carry2

            lax.fori_loop(0, gc // 2, body, 0)
            wait(0, 0, gsem0, gsem1)
            wait(1, 1, gsem2, gsem3)
            return carry

        lax.fori_loop(0, (APT + GB - 1) // GB, group, 0)
        plsc.subcore_barrier()
        pltpu.sync_copy(acc_sh.at[pl.ds(sid * ROWS_PER_TILE, ROWS_PER_TILE)],
                        out_hbm.at[pl.ds(sid * ROWS_PER_TILE, ROWS_PER_TILE)])


BM = 256
NB = NPAD // BM


def _mm1_body(d_ref, x_ref, w_ref, o_ref):
    dinv = lax.rsqrt(d_ref[...] + 1.0)
    o_ref[...] = jnp.dot(x_ref[...], w_ref[...],
                         preferred_element_type=jnp.float32) * dinv


def _mm1(d, xp, w1):
    return pl.pallas_call(
        _mm1_body,
        grid=(NB,),
        in_specs=[
            pl.BlockSpec((BM, 1), lambda i: (i, 0)),
            pl.BlockSpec((BM, D), lambda i: (i, 0)),
            pl.BlockSpec((D, D), lambda i: (0, 0)),
        ],
        out_specs=pl.BlockSpec((BM, D), lambda i: (i, 0)),
        out_shape=jax.ShapeDtypeStruct((NPAD, D), jnp.float32),
    )(d, xp, w1)


def _mm2_body(d_ref, p_ref, hs_ref, w_ref, o_ref):
    dinv = lax.rsqrt(d_ref[...] + 1.0)
    h = jnp.maximum((p_ref[...] + hs_ref[...]) * dinv, 0.0)
    o_ref[...] = jnp.dot(h, w_ref[...],
                         preferred_element_type=jnp.float32) * dinv


def _mm2(d, p, hs1, w2):
    return pl.pallas_call(
        _mm2_body,
        grid=(NB,),
        in_specs=[
            pl.BlockSpec((BM, 1), lambda i: (i, 0)),
            pl.BlockSpec((BM, D), lambda i: (i, 0)),
            pl.BlockSpec((BM, D), lambda i: (i, 0)),
            pl.BlockSpec((D, D), lambda i: (0, 0)),
        ],
        out_specs=pl.BlockSpec((BM, D), lambda i: (i, 0)),
        out_shape=jax.ShapeDtypeStruct((NPAD, D), jnp.float32),
    )(d, p, hs1, w2)


def _pool_body(d_ref, p_ref, hs_ref, b_ref, wm1_ref,
               bm1_ref, wm2_ref, bm2_ref, o_ref, pooled_acc, cnt_acc):
    i = pl.program_id(0)

    @pl.when(i == 0)
    def _():
        pooled_acc[...] = jnp.zeros((G, D), jnp.float32)
        cnt_acc[...] = jnp.zeros((G, D), jnp.float32)

    dinv = lax.rsqrt(d_ref[...] + 1.0)
    h2 = (p_ref[...] + hs_ref[...]) * dinv
    ids = jnp.broadcast_to(b_ref[...], (G, BM))
    oht = (lax.broadcasted_iota(jnp.int32, (G, BM), 0) == ids)
    oht = oht.astype(jnp.float32)
    pooled_acc[...] += jnp.dot(oht, h2, preferred_element_type=jnp.float32)
    cnt_acc[...] += jnp.dot(oht, jnp.ones((BM, D), jnp.float32),
                            preferred_element_type=jnp.float32)

    @pl.when(i == NB - 1)
    def _():
        mean = pooled_acc[...] / jnp.maximum(cnt_acc[...], 1.0)
        z = jnp.maximum(
            jnp.dot(mean, wm1_ref[...], preferred_element_type=jnp.float32)
            + bm1_ref[...], 0.0)
        o_ref[...] = jnp.dot(z, wm2_ref[...],
                             preferred_element_type=jnp.float32) + bm2_ref[...]


def _pool(d, p, hs2, batchr, wm1, bm1, wm2, bm2):
    return pl.pallas_call(
        _pool_body,
        grid=(NB,),
        in_specs=[
            pl.BlockSpec((BM, 1), lambda i: (i, 0)),
            pl.BlockSpec((BM, D), lambda i: (i, 0)),
            pl.BlockSpec((BM, D), lambda i: (i, 0)),
            pl.BlockSpec((1, BM), lambda i: (0, i)),
            pl.BlockSpec((D, D), lambda i: (0, 0)),
            pl.BlockSpec((1, D), lambda i: (0, 0)),
            pl.BlockSpec((D, D), lambda i: (0, 0)),
            pl.BlockSpec((1, D), lambda i: (0, 0)),
        ],
        out_specs=pl.BlockSpec((G, D), lambda i: (0, 0)),
        out_shape=jax.ShapeDtypeStruct((G, D), jnp.float32),
        scratch_shapes=[
            pltpu.VMEM((G, D), jnp.float32),
            pltpu.VMEM((G, D), jnp.float32),
        ],
    )(d, p, hs2, batchr, wm1, bm1, wm2, bm2)


def kernel(x, edge_index, batch, W1, W2, Wm1, bm1, Wm2, bm2):
    xp = jnp.pad(x, ((0, NPAD - N), (0, 0)))
    def _split(idx):
        a = jnp.pad(idx.astype(jnp.int32), (0, NS * APT * CH - E),
                    constant_values=N).reshape(NS, APT, CH)
        return jnp.pad(a, ((0, 0), (0, APAD - APT), (0, 0)),
                       constant_values=N)

    src = _split(edge_index[0])
    dst = _split(edge_index[1])
    batchr = jnp.pad(batch.astype(jnp.int32), (0, NPAD - N),
                     constant_values=G).reshape(1, NPAD)

    deg = _deg(dst)
    d = deg.reshape(NPAD, 1)

    hs1 = _mm1(d, xp, W1)
    p = _scatter(hs1, src, dst)
    hs2 = _mm2(d, p, hs1, W2)
    q = _scatter(hs2, src, dst)
    return _pool(d, q, hs2, batchr,
                 Wm1, bm1.reshape(1, D), Wm2, bm2.reshape(1, D))


# speedup vs baseline: 12.2971x; 1.0052x over previous
import functools
import jax
import jax.numpy as jnp
from jax import lax
from jax.experimental import pallas as pl
from jax.experimental.pallas import tpu as pltpu
from jax.experimental.pallas import tpu_sc as plsc

N = 10000
E = 320000
D = 128
G = 64

NC = 2
NS = 16
NW = NC * NS

NPAD = 10240
ROWS_PER_TILE = NPAD // NS
CH = 128
APT = 158
GB = 16
APAD = ((APT + GB - 1) // GB) * GB

_mesh = plsc.VectorSubcoreMesh(
    core_axis_name="c", subcore_axis_name="s", num_cores=NC, num_subcores=NS)


@functools.partial(
    pl.kernel,
    out_type=jax.ShapeDtypeStruct((NPAD,), jnp.float32),
    mesh=_mesh,
    scratch_types=[
        pltpu.VMEM((APAD, CH), jnp.int32),
        pltpu.VMEM((CH,), jnp.float32),
        pltpu.VMEM_SHARED((NPAD,), jnp.float32),
    ],
)
def _deg(dst_hbm, deg_out, dst_v, ones_v, deg_sh):
    cid = lax.axis_index("c")
    sid = lax.axis_index("s")

    @pl.when(cid == 1)
    def _():
        z16 = jnp.zeros((16,), jnp.float32)
        for i in range(CH // 16):
            ones_v[pl.ds(i * 16, 16)] = z16
        for i in range(ROWS_PER_TILE // CH):
            pltpu.sync_copy(
                ones_v, deg_sh.at[pl.ds(sid * ROWS_PER_TILE + i * CH, CH)])
        o16 = jnp.full((16,), 1.0, jnp.float32)
        for i in range(CH // 16):
            ones_v[pl.ds(i * 16, 16)] = o16
        pltpu.sync_copy(dst_hbm.at[sid], dst_v)
        plsc.subcore_barrier()

        def body(j, carry):
            pltpu.sync_copy(ones_v, deg_sh.at[dst_v.at[j]], add=True)
            return carry

        lax.fori_loop(0, APT, body, 0)
        plsc.subcore_barrier()
        pltpu.sync_copy(deg_sh.at[pl.ds(sid * ROWS_PER_TILE, ROWS_PER_TILE)],
                        deg_out.at[pl.ds(sid * ROWS_PER_TILE, ROWS_PER_TILE)])


@functools.partial(
    pl.kernel,
    out_type=jax.ShapeDtypeStruct((NPAD, D), jnp.float32),
    mesh=_mesh,
    scratch_types=[
        pltpu.VMEM((GB, CH), jnp.int32),
        pltpu.VMEM((GB, CH), jnp.int32),
        pltpu.VMEM((2, CH, D), jnp.float32),
        pltpu.VMEM_SHARED((NPAD, D), jnp.float32),
        pltpu.SemaphoreType.DMA,
        pltpu.SemaphoreType.DMA,
    ],
)
def _scatter(hs_hbm, src_hbm, dst_hbm, out_hbm, srcb, dstb, rows, acc_sh,
             gsem0, gsem1):
    cid = lax.axis_index("c")
    sid = lax.axis_index("s")

    @pl.when(cid == 1)
    def _():
        z16 = jnp.zeros((16,), jnp.float32)

        def zero_rows(r, carry):
            for c in range(D // 16):
                rows[0, r, pl.ds(c * 16, 16)] = z16
            return carry

        lax.fori_loop(0, CH, zero_rows, 0)

        def zinit(i, carry):
            pltpu.sync_copy(rows.at[0],
                            acc_sh.at[pl.ds(sid * ROWS_PER_TILE + i * CH, CH)])
            return carry

        lax.fori_loop(0, ROWS_PER_TILE // CH, zinit, 0)
        plsc.subcore_barrier()

        def group(g, carry):
            gc = jnp.minimum(APT - g * GB, GB)
            pltpu.sync_copy(src_hbm.at[sid, pl.ds(g * GB, GB)], srcb)
            pltpu.sync_copy(dst_hbm.at[sid, pl.ds(g * GB, GB)], dstb)
            pltpu.async_copy(hs_hbm.at[srcb.at[0]], rows.at[0], gsem0)
            pltpu.async_copy(hs_hbm.at[srcb.at[1]], rows.at[1], gsem1)

            def body(jj, carry2):
                j0 = 2 * jj
                j1 = j0 + 1
                pltpu.make_async_copy(hs_hbm.at[srcb.at[j0]], rows.at[0],
                                      gsem0).wait()
                pltpu.sync_copy(rows.at[0], acc_sh.at[dstb.at[j0]], add=True)
                pltpu.async_copy(hs_hbm.at[srcb.at[lax.rem(j0 + 2, gc)]],
                                 rows.at[0], gsem0)
                pltpu.make_async_copy(hs_hbm.at[srcb.at[j1]], rows.at[1],
                                      gsem1).wait()
                pltpu.sync_copy(rows.at[1], acc_sh.at[dstb.at[j1]], add=True)
                pltpu.async_copy(hs_hbm.at[srcb.at[lax.rem(j1 + 2, gc)]],
                                 rows.at[1], gsem1)
                return carry2

            lax.fori_loop(0, gc // 2, body, 0)
            pltpu.make_async_copy(hs_hbm.at[srcb.at[0]], rows.at[0],
                                  gsem0).wait()
            pltpu.make_async_copy(hs_hbm.at[srcb.at[1]], rows.at[1],
                                  gsem1).wait()
            return carry

        lax.fori_loop(0, (APT + GB - 1) // GB, group, 0)
        plsc.subcore_barrier()
        pltpu.sync_copy(acc_sh.at[pl.ds(sid * ROWS_PER_TILE, ROWS_PER_TILE)],
                        out_hbm.at[pl.ds(sid * ROWS_PER_TILE, ROWS_PER_TILE)])


BM = 256
NB = NPAD // BM


def _mm1_body(d_ref, x_ref, w_ref, o_ref):
    dinv = lax.rsqrt(d_ref[...] + 1.0)
    o_ref[...] = jnp.dot(x_ref[...], w_ref[...],
                         preferred_element_type=jnp.float32) * dinv


def _mm1(d, xp, w1):
    return pl.pallas_call(
        _mm1_body,
        grid=(NB,),
        in_specs=[
            pl.BlockSpec((BM, 1), lambda i: (i, 0)),
            pl.BlockSpec((BM, D), lambda i: (i, 0)),
            pl.BlockSpec((D, D), lambda i: (0, 0)),
        ],
        out_specs=pl.BlockSpec((BM, D), lambda i: (i, 0)),
        out_shape=jax.ShapeDtypeStruct((NPAD, D), jnp.float32),
    )(d, xp, w1)


def _mm2_body(d_ref, p_ref, hs_ref, w_ref, o_ref):
    dinv = lax.rsqrt(d_ref[...] + 1.0)
    h = jnp.maximum((p_ref[...] + hs_ref[...]) * dinv, 0.0)
    o_ref[...] = jnp.dot(h, w_ref[...],
                         preferred_element_type=jnp.float32) * dinv


def _mm2(d, p, hs1, w2):
    return pl.pallas_call(
        _mm2_body,
        grid=(NB,),
        in_specs=[
            pl.BlockSpec((BM, 1), lambda i: (i, 0)),
            pl.BlockSpec((BM, D), lambda i: (i, 0)),
            pl.BlockSpec((BM, D), lambda i: (i, 0)),
            pl.BlockSpec((D, D), lambda i: (0, 0)),
        ],
        out_specs=pl.BlockSpec((BM, D), lambda i: (i, 0)),
        out_shape=jax.ShapeDtypeStruct((NPAD, D), jnp.float32),
    )(d, p, hs1, w2)


def _pool_body(d_ref, p_ref, hs_ref, b_ref, wm1_ref,
               bm1_ref, wm2_ref, bm2_ref, o_ref, pooled_acc, cnt_acc):
    i = pl.program_id(0)

    @pl.when(i == 0)
    def _():
        pooled_acc[...] = jnp.zeros((G, D), jnp.float32)
        cnt_acc[...] = jnp.zeros((G, D), jnp.float32)

    dinv = lax.rsqrt(d_ref[...] + 1.0)
    h2 = (p_ref[...] + hs_ref[...]) * dinv
    ids = jnp.broadcast_to(b_ref[...], (G, BM))
    oht = (lax.broadcasted_iota(jnp.int32, (G, BM), 0) == ids)
    oht = oht.astype(jnp.float32)
    pooled_acc[...] += jnp.dot(oht, h2, preferred_element_type=jnp.float32)
    cnt_acc[...] += jnp.dot(oht, jnp.ones((BM, D), jnp.float32),
                            preferred_element_type=jnp.float32)

    @pl.when(i == NB - 1)
    def _():
        mean = pooled_acc[...] / jnp.maximum(cnt_acc[...], 1.0)
        z = jnp.maximum(
            jnp.dot(mean, wm1_ref[...], preferred_element_type=jnp.float32)
            + bm1_ref[...], 0.0)
        o_ref[...] = jnp.dot(z, wm2_ref[...],
                             preferred_element_type=jnp.float32) + bm2_ref[...]


def _pool(d, p, hs2, batchr, wm1, bm1, wm2, bm2):
    return pl.pallas_call(
        _pool_body,
        grid=(NB,),
        in_specs=[
            pl.BlockSpec((BM, 1), lambda i: (i, 0)),
            pl.BlockSpec((BM, D), lambda i: (i, 0)),
            pl.BlockSpec((BM, D), lambda i: (i, 0)),
            pl.BlockSpec((1, BM), lambda i: (0, i)),
            pl.BlockSpec((D, D), lambda i: (0, 0)),
            pl.BlockSpec((1, D), lambda i: (0, 0)),
            pl.BlockSpec((D, D), lambda i: (0, 0)),
            pl.BlockSpec((1, D), lambda i: (0, 0)),
        ],
        out_specs=pl.BlockSpec((G, D), lambda i: (0, 0)),
        out_shape=jax.ShapeDtypeStruct((G, D), jnp.float32),
        scratch_shapes=[
            pltpu.VMEM((G, D), jnp.float32),
            pltpu.VMEM((G, D), jnp.float32),
        ],
    )(d, p, hs2, batchr, wm1, bm1, wm2, bm2)


def kernel(x, edge_index, batch, W1, W2, Wm1, bm1, Wm2, bm2):
    xp = jnp.pad(x, ((0, NPAD - N), (0, 0)))
    def _split(idx):
        a = jnp.pad(idx.astype(jnp.int32), (0, NS * APT * CH - E),
                    constant_values=N).reshape(NS, APT, CH)
        return jnp.pad(a, ((0, 0), (0, APAD - APT), (0, 0)),
                       constant_values=N)

    src = _split(edge_index[0])
    dst = _split(edge_index[1])
    batchr = jnp.pad(batch.astype(jnp.int32), (0, NPAD - N),
                     constant_values=G).reshape(1, NPAD)

    deg = _deg(dst)
    d = deg.reshape(NPAD, 1)

    hs1 = _mm1(d, xp, W1)
    p = _scatter(hs1, src, dst)
    hs2 = _mm2(d, p, hs1, W2)
    q = _scatter(hs2, src, dst)
    return _pool(d, q, hs2, batchr,
                 Wm1, bm1.reshape(1, D), Wm2, bm2.reshape(1, D))
